# baseline (speedup 1.0000x reference)
import functools
import numpy as np
import jax
import jax.numpy as jnp
from jax import lax
from jax.experimental import pallas as pl
from jax.experimental.pallas import tpu as pltpu
from jax.experimental.pallas import tpu_sc as plsc

N, E, D, DE, H = 10000, 160000, 128, 16, 256

_SC_NC, _SC_NS = 2, 16
_NW = _SC_NC * _SC_NS
_EPW = E // _NW
_C = 40
_NCHUNK = _EPW // _C


_PREC = lax.Precision.HIGHEST


def _pack_bf16(acc):
    hw = acc.shape[1] // 2
    lo = lax.bitcast_convert_type(
        acc[:, :hw].astype(jnp.bfloat16), jnp.uint16).astype(jnp.uint32)
    hi = lax.bitcast_convert_type(
        acc[:, hw:].astype(jnp.bfloat16), jnp.uint16).astype(jnp.uint32)
    return lax.bitcast_convert_type(lo | (hi << 16), jnp.int32)


def _mm_body(relu, pack, n_out, a_ref, *refs):
    a = a_ref[...]
    w_refs = refs[:n_out]
    b_refs = refs[n_out:2 * n_out]
    o_refs = refs[2 * n_out:]
    for w_ref, b_ref, o_ref, pk in zip(w_refs, b_refs, o_refs, pack):
        acc = jnp.dot(a, w_ref[...], preferred_element_type=jnp.float32,
                      precision=_PREC) + b_ref[...]
        if relu:
            acc = jnp.maximum(acc, 0.0)
        if pk:
            o_ref[...] = _pack_bf16(acc)
        elif len(o_ref.shape) == 3:
            hw = o_ref.shape[-1]
            o_ref[0] = acc[:, :hw]
            o_ref[1] = acc[:, hw:]
        else:
            o_ref[...] = acc.astype(o_ref.dtype)


def _mm(a, ws, bs, block, relu=False, pack=None, split=None):
    M, K = a.shape
    n_out = len(ws)
    if pack is None:
        pack = [False] * n_out
    if split is None:
        split = [False] * n_out
    grid = (M // block,)
    in_specs = [pl.BlockSpec((block, K), lambda i: (i, 0))]
    in_specs += [pl.BlockSpec(w.shape, lambda i: (0, 0)) for w in ws]
    in_specs += [pl.BlockSpec(b.shape, lambda i: (0,)) for b in bs]
    out_specs = []
    out_shape = []
    for w, pk, sp in zip(ws, pack, split):
        ki = w.shape[1]
        if sp:
            out_specs.append(pl.BlockSpec((2, block, ki // 2),
                                          lambda i: (0, i, 0)))
            out_shape.append(jax.ShapeDtypeStruct((2, M, ki // 2), jnp.float32))
        else:
            kk = ki // 2 if pk else ki
            out_specs.append(pl.BlockSpec((block, kk), lambda i: (i, 0)))
            out_shape.append(jax.ShapeDtypeStruct(
                (M, kk), jnp.int32 if pk else jnp.float32))
    return pl.pallas_call(
        functools.partial(_mm_body, relu, pack, n_out),
        grid=grid,
        in_specs=in_specs,
        out_specs=out_specs,
        out_shape=out_shape,
    )(a, *ws, *bs)


def _node_update_body(x_ref, aggw_ref, agg2_ref, g_ref,
                      wx_ref, wa2_ref, bn_ref, wxs_ref, wxd_ref,
                      oxs_ref, oxd_ref):
    dot = functools.partial(jnp.dot, preferred_element_type=jnp.float32,
                            precision=_PREC)
    v = (dot(x_ref[...], wx_ref[...]) + aggw_ref[...]
         + dot(agg2_ref[...], wa2_ref[...]) + g_ref[...] + bn_ref[...])
    v = jnp.maximum(v, 0.0)
    oxs_ref[...] = _pack_bf16(dot(v, wxs_ref[...]))
    oxd_ref[...] = _pack_bf16(dot(v, wxd_ref[...]))


def _node_update(x, aggw, agg2, g, wx, wa2, bn, wxs, wxd, block=1000):
    grid = (N // block,)
    row = lambda k: pl.BlockSpec((block, k), lambda i: (i, 0))
    full = lambda w: pl.BlockSpec(w.shape, lambda i: (0,) * w.ndim)
    return pl.pallas_call(
        _node_update_body,
        grid=grid,
        in_specs=[row(D), row(D), row(DE), row(D),
                  full(wx), full(wa2), full(bn), full(wxs), full(wxd)],
        out_specs=[row(H // 2), row(H // 2)],
        out_shape=[jax.ShapeDtypeStruct((N, H // 2), jnp.int32)] * 2,
    )(x, aggw, agg2, g, wx, wa2, bn, wxs, wxd)


def _edge_sum_body(tx_hbm, td_hbm, ix_hbm, id_hbm, pe_hbm, out_hbm,
                   ixbs, idbs, gxs, gds, pebs, ob, sems):
    cid = lax.axis_index("c")
    sid = lax.axis_index("s")
    wid = sid * _SC_NC + cid
    ebase = wid * _EPW

    def fire(k, b):
        o = ebase + k * _C
        pltpu.sync_copy(ix_hbm.at[pl.ds(o * 2, 2 * _C)], ixbs[b])
        pltpu.sync_copy(id_hbm.at[pl.ds(o * 3, 3 * _C)], idbs[b])
        pltpu.async_copy(tx_hbm.at[ixbs[b]], gxs[b], sems[b])
        pltpu.async_copy(td_hbm.at[idbs[b]], gds[b], sems[b])
        pltpu.async_copy(pe_hbm.at[pl.ds(o, _C)], pebs[b], sems[b])

    def wait(b):
        pltpu.make_async_copy(tx_hbm.at[pl.ds(0, 2 * _C)], gxs[b], sems[b]).wait()
        pltpu.make_async_copy(td_hbm.at[pl.ds(0, 3 * _C)], gds[b], sems[b]).wait()
        pltpu.make_async_copy(pe_hbm.at[pl.ds(0, _C)], pebs[b], sems[b]).wait()

    def split(w):
        a = plsc.bitcast(w << 16, jnp.float32)
        b = plsc.bitcast(w & jnp.int32(-65536), jnp.float32)
        return a, b

    def compute(k, b):
        gx, gd, peb = gxs[b], gds[b], pebs[b]

        def row(e, c2):
            for j in range(8):
                s = pl.ds(j * 16, 16)
                pa, pb = split(peb[e, s])
                for r in (gx.at[2 * e], gx.at[2 * e + 1], gd.at[3 * e],
                          gd.at[3 * e + 1], gd.at[3 * e + 2]):
                    ua, ub = split(r[s])
                    pa = pa + ua
                    pb = pb + ub
                base = j * 16
                ob[e, pl.ds(base, 16)] = jnp.maximum(pa, 0.0)
                ob[e, pl.ds(base + 128, 16)] = jnp.maximum(pb, 0.0)
            return c2

        lax.fori_loop(0, _C, row, 0)
        pltpu.sync_copy(ob, out_hbm.at[pl.ds(ebase + k * _C, _C)])

    fire(0, 0)

    def pair(k2, carry):
        k = 2 * k2
        fire(k + 1, 1)
        wait(0)
        compute(k, 0)
        fire(k + 2, 0)
        wait(1)
        compute(k + 1, 1)
        return carry

    lax.fori_loop(0, (_NCHUNK - 1) // 2, pair, 0)
    wait(0)
    compute(_NCHUNK - 1, 0)


def _edge_sum(tx, td, idx_x, idx_d, pe):
    mesh = plsc.VectorSubcoreMesh(core_axis_name="c", subcore_axis_name="s",
                                  num_cores=_SC_NC, num_subcores=_SC_NS)
    f = pl.kernel(
        _edge_sum_body,
        out_type=jax.ShapeDtypeStruct((E, H), jnp.float32),
        mesh=mesh,
        compiler_params=pltpu.CompilerParams(needs_layout_passes=False),
        scratch_types=[
            [pltpu.VMEM((2 * _C,), jnp.int32)] * 2,
            [pltpu.VMEM((3 * _C,), jnp.int32)] * 2,
            [pltpu.VMEM((2 * _C, 128), jnp.int32)] * 2,
            [pltpu.VMEM((3 * _C, 128), jnp.int32)] * 2,
            [pltpu.VMEM((_C, 128), jnp.int32)] * 2,
            pltpu.VMEM((_C, H), jnp.float32),
            [pltpu.SemaphoreType.DMA] * 2,
        ],
    )
    return f(tx, td, idx_x, idx_d, pe)


_C2T = 80
_C2 = 40


def kernel(x, edge_index, edge_index_NODES, edge_attr, det_features,
           temp_indices_NODES, temp_indices_EDGES,
           first_det_indices_NODES, first_det_indices_EDGES,
           second_det_indices_NODES, second_det_indices_EDGES,
           indices_for_aggregating_nodes_updates,
           W_e1, b_e1, W_e2, b_e2, W_n, b_n):
    src, dst = edge_index[0], edge_index[1]
    dstN = edge_index_NODES[1]
    fE, sE, tE = (first_det_indices_EDGES, second_det_indices_EDGES,
                  temp_indices_EDGES)
    fN, sN, tN = (first_det_indices_NODES, second_det_indices_NODES,
                  temp_indices_NODES)
    agg_idx = indices_for_aggregating_nodes_updates

    W1_xs, W1_xd = W_e1[0:D], W_e1[D:2*D]
    W1_e = W_e1[2*D:2*D+DE]
    W1_f, W1_s, W1_t = (W_e1[2*D+DE:3*D+DE], W_e1[3*D+DE:4*D+DE],
                        W_e1[4*D+DE:5*D+DE])
    Wn_x, Wn_a, Wn_a2 = W_n[0:D], W_n[D:D+H], W_n[D+H:D+H+DE]
    Wn_t, Wn_f, Wn_s = (W_n[D+H+DE:2*D+H+DE], W_n[2*D+H+DE:3*D+H+DE],
                        W_n[3*D+H+DE:4*D+H+DE])
    zH = jnp.zeros((H,), jnp.float32)
    zD = jnp.zeros((D,), jnp.float32)
    zE = jnp.zeros((DE,), jnp.float32)

    Pdf, Pds, Pdt, Qt, Qf, Qs = _mm(
        det_features, [W1_f, W1_s, W1_t, Wn_t, Wn_f, Wn_s],
        [zH, zH, zH, zD, zD, zD], block=1000,
        pack=[True, True, True, False, False, False])
    Pxs, Pxd = _mm(x, [W1_xs, W1_xd], [zH, zH], block=1000, pack=[True, True])
    (Pe1,) = _mm(edge_attr, [W1_e], [b_e1], block=1600, pack=[True])

    Td = jnp.concatenate([Pdf, Pds, Pdt], axis=0)
    idx_x = jnp.stack([src, dst + N], axis=1).reshape(-1).astype(jnp.int32)
    idx_d = jnp.stack([fE, sE + N, tE + 2 * N], axis=1).reshape(-1).astype(jnp.int32)

    Tx1 = jnp.concatenate([Pxs, Pxd], axis=0)
    h1 = _edge_sum(Tx1, Td, idx_x, idx_d, Pe1)

    Wf = (W_e2 @ W1_e).astype(jnp.float32)
    cf = (b_e2 @ W1_e + b_e1).astype(jnp.float32)
    e1, Pe2, t = _mm(h1, [W_e2, Wf, Wn_a], [b_e2, cf, zD], block=1600,
                     pack=[False, True, False])

    aggw = jax.ops.segment_sum(t, agg_idx, num_segments=N)
    agg2 = jax.ops.segment_sum(e1, dstN, num_segments=N)
    G = Qt[tN] + Qf[fN] + Qs[sN]

    Px2s, Px2d = _node_update(x, aggw, agg2, G, Wn_x, Wn_a2, b_n,
                              W1_xs, W1_xd)

    Tx2 = jnp.concatenate([Px2s, Px2d], axis=0)
    h2 = _edge_sum(Tx2, Td, idx_x, idx_d, Pe2)
    (e2,) = _mm(h2, [W_e2], [b_e2], block=1600)
    return e2


# speedup vs baseline: 1.0630x; 1.0630x over previous
import functools
import numpy as np
import jax
import jax.numpy as jnp
from jax import lax
from jax.experimental import pallas as pl
from jax.experimental.pallas import tpu as pltpu
from jax.experimental.pallas import tpu_sc as plsc

N, E, D, DE, H = 10000, 160000, 128, 16, 256

_SC_NC, _SC_NS = 2, 16
_NW = _SC_NC * _SC_NS
_EPW = E // _NW
_C = 40
_NCHUNK = _EPW // _C


_PREC = lax.Precision.DEFAULT


def _pack_bf16(acc):
    hw = acc.shape[1] // 2
    lo = lax.bitcast_convert_type(
        acc[:, :hw].astype(jnp.bfloat16), jnp.uint16).astype(jnp.uint32)
    hi = lax.bitcast_convert_type(
        acc[:, hw:].astype(jnp.bfloat16), jnp.uint16).astype(jnp.uint32)
    return lax.bitcast_convert_type(lo | (hi << 16), jnp.int32)


def _mm_body(relu, pack, n_out, a_ref, *refs):
    a = a_ref[...]
    w_refs = refs[:n_out]
    b_refs = refs[n_out:2 * n_out]
    o_refs = refs[2 * n_out:]
    for w_ref, b_ref, o_ref, pk in zip(w_refs, b_refs, o_refs, pack):
        acc = jnp.dot(a, w_ref[...], preferred_element_type=jnp.float32,
                      precision=_PREC) + b_ref[...]
        if relu:
            acc = jnp.maximum(acc, 0.0)
        if pk:
            o_ref[...] = _pack_bf16(acc)
        elif len(o_ref.shape) == 3:
            hw = o_ref.shape[-1]
            o_ref[0] = acc[:, :hw]
            o_ref[1] = acc[:, hw:]
        else:
            o_ref[...] = acc.astype(o_ref.dtype)


def _mm(a, ws, bs, block, relu=False, pack=None, split=None):
    M, K = a.shape
    n_out = len(ws)
    if pack is None:
        pack = [False] * n_out
    if split is None:
        split = [False] * n_out
    grid = (M // block,)
    in_specs = [pl.BlockSpec((block, K), lambda i: (i, 0))]
    in_specs += [pl.BlockSpec(w.shape, lambda i: (0, 0)) for w in ws]
    in_specs += [pl.BlockSpec(b.shape, lambda i: (0,)) for b in bs]
    out_specs = []
    out_shape = []
    for w, pk, sp in zip(ws, pack, split):
        ki = w.shape[1]
        if sp:
            out_specs.append(pl.BlockSpec((2, block, ki // 2),
                                          lambda i: (0, i, 0)))
            out_shape.append(jax.ShapeDtypeStruct((2, M, ki // 2), jnp.float32))
        else:
            kk = ki // 2 if pk else ki
            out_specs.append(pl.BlockSpec((block, kk), lambda i: (i, 0)))
            out_shape.append(jax.ShapeDtypeStruct(
                (M, kk), jnp.int32 if pk else jnp.float32))
    return pl.pallas_call(
        functools.partial(_mm_body, relu, pack, n_out),
        grid=grid,
        in_specs=in_specs,
        out_specs=out_specs,
        out_shape=out_shape,
    )(a, *ws, *bs)


def _node_update_body(x_ref, agg_ref, agg2_ref, g_ref,
                      wx_ref, wa_ref, wa2_ref, bn_ref, wxs_ref, wxd_ref,
                      oxs_ref, oxd_ref):
    dot = functools.partial(jnp.dot, preferred_element_type=jnp.float32,
                            precision=_PREC)
    v = (dot(x_ref[...], wx_ref[...]) + dot(agg_ref[...], wa_ref[...])
         + dot(agg2_ref[...], wa2_ref[...]) + g_ref[...] + bn_ref[...])
    v = jnp.maximum(v, 0.0)
    oxs_ref[...] = _pack_bf16(dot(v, wxs_ref[...]))
    oxd_ref[...] = _pack_bf16(dot(v, wxd_ref[...]))


def _node_update(x, agg, agg2, g, wx, wa, wa2, bn, wxs, wxd, block=1000):
    grid = (N // block,)
    row = lambda k: pl.BlockSpec((block, k), lambda i: (i, 0))
    full = lambda w: pl.BlockSpec(w.shape, lambda i: (0,) * w.ndim)
    return pl.pallas_call(
        _node_update_body,
        grid=grid,
        in_specs=[row(D), row(H), row(DE), row(D),
                  full(wx), full(wa), full(wa2), full(bn), full(wxs), full(wxd)],
        out_specs=[row(H // 2), row(H // 2)],
        out_shape=[jax.ShapeDtypeStruct((N, H // 2), jnp.int32)] * 2,
    )(x, agg, agg2, g, wx, wa, wa2, bn, wxs, wxd)


def _edge_sum_body(tx_hbm, td_hbm, ix_hbm, id_hbm, pe_hbm, out_hbm,
                   ixbs, idbs, gxs, gds, pebs, ob, sems):
    cid = lax.axis_index("c")
    sid = lax.axis_index("s")
    wid = sid * _SC_NC + cid
    ebase = wid * _EPW

    def fire(k, b):
        o = ebase + k * _C
        pltpu.sync_copy(ix_hbm.at[pl.ds(o * 2, 2 * _C)], ixbs[b])
        pltpu.sync_copy(id_hbm.at[pl.ds(o * 3, 3 * _C)], idbs[b])
        pltpu.async_copy(tx_hbm.at[ixbs[b]], gxs[b], sems[b])
        pltpu.async_copy(td_hbm.at[idbs[b]], gds[b], sems[b])
        pltpu.async_copy(pe_hbm.at[pl.ds(o, _C)], pebs[b], sems[b])

    def wait(b):
        pltpu.make_async_copy(tx_hbm.at[pl.ds(0, 2 * _C)], gxs[b], sems[b]).wait()
        pltpu.make_async_copy(td_hbm.at[pl.ds(0, 3 * _C)], gds[b], sems[b]).wait()
        pltpu.make_async_copy(pe_hbm.at[pl.ds(0, _C)], pebs[b], sems[b]).wait()

    def split(w):
        a = plsc.bitcast(w << 16, jnp.float32)
        b = plsc.bitcast(w & jnp.int32(-65536), jnp.float32)
        return a, b

    def compute(k, b):
        gx, gd, peb = gxs[b], gds[b], pebs[b]

        def row(e, c2):
            for j in range(8):
                s = pl.ds(j * 16, 16)
                pa, pb = split(peb[e, s])
                for r in (gx.at[2 * e], gx.at[2 * e + 1], gd.at[3 * e],
                          gd.at[3 * e + 1], gd.at[3 * e + 2]):
                    ua, ub = split(r[s])
                    pa = pa + ua
                    pb = pb + ub
                base = j * 16
                ob[e, pl.ds(base, 16)] = jnp.maximum(pa, 0.0)
                ob[e, pl.ds(base + 128, 16)] = jnp.maximum(pb, 0.0)
            return c2

        lax.fori_loop(0, _C, row, 0)
        pltpu.sync_copy(ob, out_hbm.at[pl.ds(ebase + k * _C, _C)])

    fire(0, 0)

    def pair(k2, carry):
        k = 2 * k2
        fire(k + 1, 1)
        wait(0)
        compute(k, 0)
        fire(k + 2, 0)
        wait(1)
        compute(k + 1, 1)
        return carry

    lax.fori_loop(0, (_NCHUNK - 1) // 2, pair, 0)
    wait(0)
    compute(_NCHUNK - 1, 0)


def _edge_sum(tx, td, idx_x, idx_d, pe):
    mesh = plsc.VectorSubcoreMesh(core_axis_name="c", subcore_axis_name="s",
                                  num_cores=_SC_NC, num_subcores=_SC_NS)
    f = pl.kernel(
        _edge_sum_body,
        out_type=jax.ShapeDtypeStruct((E, H), jnp.float32),
        mesh=mesh,
        compiler_params=pltpu.CompilerParams(needs_layout_passes=False),
        scratch_types=[
            [pltpu.VMEM((2 * _C,), jnp.int32)] * 2,
            [pltpu.VMEM((3 * _C,), jnp.int32)] * 2,
            [pltpu.VMEM((2 * _C, 128), jnp.int32)] * 2,
            [pltpu.VMEM((3 * _C, 128), jnp.int32)] * 2,
            [pltpu.VMEM((_C, 128), jnp.int32)] * 2,
            pltpu.VMEM((_C, H), jnp.float32),
            [pltpu.SemaphoreType.DMA] * 2,
        ],
    )
    return f(tx, td, idx_x, idx_d, pe)


_C2T = 80
_C2 = 40


def kernel(x, edge_index, edge_index_NODES, edge_attr, det_features,
           temp_indices_NODES, temp_indices_EDGES,
           first_det_indices_NODES, first_det_indices_EDGES,
           second_det_indices_NODES, second_det_indices_EDGES,
           indices_for_aggregating_nodes_updates,
           W_e1, b_e1, W_e2, b_e2, W_n, b_n):
    src, dst = edge_index[0], edge_index[1]
    dstN = edge_index_NODES[1]
    fE, sE, tE = (first_det_indices_EDGES, second_det_indices_EDGES,
                  temp_indices_EDGES)
    fN, sN, tN = (first_det_indices_NODES, second_det_indices_NODES,
                  temp_indices_NODES)
    agg_idx = indices_for_aggregating_nodes_updates

    W1_xs, W1_xd = W_e1[0:D], W_e1[D:2*D]
    W1_e = W_e1[2*D:2*D+DE]
    W1_f, W1_s, W1_t = (W_e1[2*D+DE:3*D+DE], W_e1[3*D+DE:4*D+DE],
                        W_e1[4*D+DE:5*D+DE])
    Wn_x, Wn_a, Wn_a2 = W_n[0:D], W_n[D:D+H], W_n[D+H:D+H+DE]
    Wn_t, Wn_f, Wn_s = (W_n[D+H+DE:2*D+H+DE], W_n[2*D+H+DE:3*D+H+DE],
                        W_n[3*D+H+DE:4*D+H+DE])
    zH = jnp.zeros((H,), jnp.float32)
    zD = jnp.zeros((D,), jnp.float32)
    zE = jnp.zeros((DE,), jnp.float32)

    Pdf, Pds, Pdt, Qt, Qf, Qs = _mm(
        det_features, [W1_f, W1_s, W1_t, Wn_t, Wn_f, Wn_s],
        [zH, zH, zH, zD, zD, zD], block=1000,
        pack=[True, True, True, False, False, False])
    Pxs, Pxd = _mm(x, [W1_xs, W1_xd], [zH, zH], block=1000, pack=[True, True])
    (Pe1,) = _mm(edge_attr, [W1_e], [b_e1], block=1600, pack=[True])

    Td = jnp.concatenate([Pdf, Pds, Pdt], axis=0)
    idx_x = jnp.stack([src, dst + N], axis=1).reshape(-1).astype(jnp.int32)
    idx_d = jnp.stack([fE, sE + N, tE + 2 * N], axis=1).reshape(-1).astype(jnp.int32)

    Tx1 = jnp.concatenate([Pxs, Pxd], axis=0)
    h1 = _edge_sum(Tx1, Td, idx_x, idx_d, Pe1)

    Wf = (W_e2 @ W1_e).astype(jnp.float32)
    cf = (b_e2 @ W1_e + b_e1).astype(jnp.float32)
    e1, Pe2 = _mm(h1, [W_e2, Wf], [b_e2, cf], block=1600,
                  pack=[False, True])

    agg = jax.ops.segment_sum(h1, agg_idx, num_segments=N)
    agg2 = jax.ops.segment_sum(e1, dstN, num_segments=N)
    G = Qt[tN] + Qf[fN] + Qs[sN]

    Px2s, Px2d = _node_update(x, agg, agg2, G, Wn_x, Wn_a, Wn_a2, b_n,
                              W1_xs, W1_xd)

    Tx2 = jnp.concatenate([Px2s, Px2d], axis=0)
    h2 = _edge_sum(Tx2, Td, idx_x, idx_d, Pe2)
    (e2,) = _mm(h2, [W_e2], [b_e2], block=1600)
    return e2


# speedup vs baseline: 1.1555x; 1.0870x over previous
import functools
import numpy as np
import jax
import jax.numpy as jnp
from jax import lax
from jax.experimental import pallas as pl
from jax.experimental.pallas import tpu as pltpu
from jax.experimental.pallas import tpu_sc as plsc

N, E, D, DE, H = 10000, 160000, 128, 16, 256

_SC_NC, _SC_NS = 2, 16
_NW = _SC_NC * _SC_NS
_EPW = E // _NW
_C = 40
_NCHUNK = _EPW // _C


_PREC = lax.Precision.DEFAULT


def _pack_bf16(acc):
    hw = acc.shape[1] // 2
    lo = lax.bitcast_convert_type(
        acc[:, :hw].astype(jnp.bfloat16), jnp.uint16).astype(jnp.uint32)
    hi = lax.bitcast_convert_type(
        acc[:, hw:].astype(jnp.bfloat16), jnp.uint16).astype(jnp.uint32)
    return lax.bitcast_convert_type(lo | (hi << 16), jnp.int32)


def _mm_body(relu, pack, n_out, a_ref, *refs):
    a = a_ref[...]
    w_refs = refs[:n_out]
    b_refs = refs[n_out:2 * n_out]
    o_refs = refs[2 * n_out:]
    for w_ref, b_ref, o_ref, pk in zip(w_refs, b_refs, o_refs, pack):
        acc = jnp.dot(a, w_ref[...], preferred_element_type=jnp.float32,
                      precision=_PREC) + b_ref[...]
        if relu:
            acc = jnp.maximum(acc, 0.0)
        if pk:
            o_ref[...] = _pack_bf16(acc)
        elif len(o_ref.shape) == 3:
            hw = o_ref.shape[-1]
            o_ref[0] = acc[:, :hw]
            o_ref[1] = acc[:, hw:]
        else:
            o_ref[...] = acc.astype(o_ref.dtype)


def _mm(a, ws, bs, block, relu=False, pack=None, split=None):
    M, K = a.shape
    n_out = len(ws)
    if pack is None:
        pack = [False] * n_out
    if split is None:
        split = [False] * n_out
    grid = (M // block,)
    in_specs = [pl.BlockSpec((block, K), lambda i: (i, 0))]
    in_specs += [pl.BlockSpec(w.shape, lambda i: (0, 0)) for w in ws]
    in_specs += [pl.BlockSpec(b.shape, lambda i: (0,)) for b in bs]
    out_specs = []
    out_shape = []
    for w, pk, sp in zip(ws, pack, split):
        ki = w.shape[1]
        if sp:
            out_specs.append(pl.BlockSpec((2, block, ki // 2),
                                          lambda i: (0, i, 0)))
            out_shape.append(jax.ShapeDtypeStruct((2, M, ki // 2), jnp.float32))
        else:
            kk = ki // 2 if pk else ki
            out_specs.append(pl.BlockSpec((block, kk), lambda i: (i, 0)))
            out_shape.append(jax.ShapeDtypeStruct(
                (M, kk), jnp.int32 if pk else jnp.float32))
    return pl.pallas_call(
        functools.partial(_mm_body, relu, pack, n_out),
        grid=grid,
        in_specs=in_specs,
        out_specs=out_specs,
        out_shape=out_shape,
    )(a, *ws, *bs)


def _node_update_body(x_ref, agg_ref, agg2_ref, g_ref,
                      wx_ref, wa_ref, wa2_ref, bn_ref, wxs_ref, wxd_ref,
                      oxs_ref, oxd_ref):
    dot = functools.partial(jnp.dot, preferred_element_type=jnp.float32,
                            precision=_PREC)
    v = (dot(x_ref[...], wx_ref[...]) + dot(agg_ref[...], wa_ref[...])
         + dot(agg2_ref[...], wa2_ref[...]) + g_ref[...] + bn_ref[...])
    v = jnp.maximum(v, 0.0)
    oxs_ref[...] = _pack_bf16(dot(v, wxs_ref[...]))
    oxd_ref[...] = _pack_bf16(dot(v, wxd_ref[...]))


def _node_update(x, agg, agg2, g, wx, wa, wa2, bn, wxs, wxd, block=1000):
    grid = (N // block,)
    row = lambda k: pl.BlockSpec((block, k), lambda i: (i, 0))
    full = lambda w: pl.BlockSpec(w.shape, lambda i: (0,) * w.ndim)
    return pl.pallas_call(
        _node_update_body,
        grid=grid,
        in_specs=[row(D), row(H), row(DE), row(D),
                  full(wx), full(wa), full(wa2), full(bn), full(wxs), full(wxd)],
        out_specs=[row(H // 2), row(H // 2)],
        out_shape=[jax.ShapeDtypeStruct((N, H // 2), jnp.int32)] * 2,
    )(x, agg, agg2, g, wx, wa, wa2, bn, wxs, wxd)


def _edge_sum_body(tx_hbm, td_hbm, ix_hbm, id_hbm, pe_hbm, out_hbm,
                   ixa, ida, gxs, gds, pebs, ob, sems):
    cid = lax.axis_index("c")
    sid = lax.axis_index("s")
    wid = sid * _SC_NC + cid
    ebase = wid * _EPW

    pltpu.sync_copy(ix_hbm.at[pl.ds(ebase * 2, _EPW * 2)], ixa)
    pltpu.sync_copy(id_hbm.at[pl.ds(ebase * 3, _EPW * 3)], ida)

    def fire(k, b):
        o = k * _C
        pltpu.async_copy(tx_hbm.at[ixa.at[pl.ds(o * 2, 2 * _C)]], gxs[b], sems[b])
        pltpu.async_copy(td_hbm.at[ida.at[pl.ds(o * 3, 3 * _C)]], gds[b], sems[b])
        pltpu.async_copy(pe_hbm.at[pl.ds(ebase + o, _C)], pebs[b], sems[b])

    def wait(b):
        pltpu.make_async_copy(tx_hbm.at[pl.ds(0, 2 * _C)], gxs[b], sems[b]).wait()
        pltpu.make_async_copy(td_hbm.at[pl.ds(0, 3 * _C)], gds[b], sems[b]).wait()
        pltpu.make_async_copy(pe_hbm.at[pl.ds(0, _C)], pebs[b], sems[b]).wait()

    def split(w):
        a = plsc.bitcast(w << 16, jnp.float32)
        b = plsc.bitcast(w & jnp.int32(-65536), jnp.float32)
        return a, b

    def compute(k, b):
        gx, gd, peb = gxs[b], gds[b], pebs[b]

        def row(e, c2):
            for j in range(8):
                s = pl.ds(j * 16, 16)
                pa, pb = split(peb[e, s])
                for r in (gx.at[2 * e], gx.at[2 * e + 1], gd.at[3 * e],
                          gd.at[3 * e + 1], gd.at[3 * e + 2]):
                    ua, ub = split(r[s])
                    pa = pa + ua
                    pb = pb + ub
                base = j * 16
                ob[e, pl.ds(base, 16)] = jnp.maximum(pa, 0.0)
                ob[e, pl.ds(base + 128, 16)] = jnp.maximum(pb, 0.0)
            return c2

        lax.fori_loop(0, _C, row, 0)
        pltpu.sync_copy(ob, out_hbm.at[pl.ds(ebase + k * _C, _C)])

    fire(0, 0)

    def pair(k2, carry):
        k = 2 * k2
        fire(k + 1, 1)
        wait(0)
        compute(k, 0)
        fire(k + 2, 0)
        wait(1)
        compute(k + 1, 1)
        return carry

    lax.fori_loop(0, (_NCHUNK - 1) // 2, pair, 0)
    wait(0)
    compute(_NCHUNK - 1, 0)


def _edge_sum(tx, td, idx_x, idx_d, pe):
    mesh = plsc.VectorSubcoreMesh(core_axis_name="c", subcore_axis_name="s",
                                  num_cores=_SC_NC, num_subcores=_SC_NS)
    f = pl.kernel(
        _edge_sum_body,
        out_type=jax.ShapeDtypeStruct((E, H), jnp.float32),
        mesh=mesh,
        compiler_params=pltpu.CompilerParams(needs_layout_passes=False),
        scratch_types=[
            pltpu.VMEM((2 * _EPW,), jnp.int32),
            pltpu.VMEM((3 * _EPW,), jnp.int32),
            [pltpu.VMEM((2 * _C, 128), jnp.int32)] * 2,
            [pltpu.VMEM((3 * _C, 128), jnp.int32)] * 2,
            [pltpu.VMEM((_C, 128), jnp.int32)] * 2,
            pltpu.VMEM((_C, H), jnp.float32),
            [pltpu.SemaphoreType.DMA] * 2,
        ],
    )
    return f(tx, td, idx_x, idx_d, pe)


_C2T = 80
_C2 = 40


def kernel(x, edge_index, edge_index_NODES, edge_attr, det_features,
           temp_indices_NODES, temp_indices_EDGES,
           first_det_indices_NODES, first_det_indices_EDGES,
           second_det_indices_NODES, second_det_indices_EDGES,
           indices_for_aggregating_nodes_updates,
           W_e1, b_e1, W_e2, b_e2, W_n, b_n):
    src, dst = edge_index[0], edge_index[1]
    dstN = edge_index_NODES[1]
    fE, sE, tE = (first_det_indices_EDGES, second_det_indices_EDGES,
                  temp_indices_EDGES)
    fN, sN, tN = (first_det_indices_NODES, second_det_indices_NODES,
                  temp_indices_NODES)
    agg_idx = indices_for_aggregating_nodes_updates

    W1_xs, W1_xd = W_e1[0:D], W_e1[D:2*D]
    W1_e = W_e1[2*D:2*D+DE]
    W1_f, W1_s, W1_t = (W_e1[2*D+DE:3*D+DE], W_e1[3*D+DE:4*D+DE],
                        W_e1[4*D+DE:5*D+DE])
    Wn_x, Wn_a, Wn_a2 = W_n[0:D], W_n[D:D+H], W_n[D+H:D+H+DE]
    Wn_t, Wn_f, Wn_s = (W_n[D+H+DE:2*D+H+DE], W_n[2*D+H+DE:3*D+H+DE],
                        W_n[3*D+H+DE:4*D+H+DE])
    zH = jnp.zeros((H,), jnp.float32)
    zD = jnp.zeros((D,), jnp.float32)
    zE = jnp.zeros((DE,), jnp.float32)

    Pdf, Pds, Pdt, Qt, Qf, Qs = _mm(
        det_features, [W1_f, W1_s, W1_t, Wn_t, Wn_f, Wn_s],
        [zH, zH, zH, zD, zD, zD], block=1000,
        pack=[True, True, True, False, False, False])
    Pxs, Pxd = _mm(x, [W1_xs, W1_xd], [zH, zH], block=1000, pack=[True, True])
    (Pe1,) = _mm(edge_attr, [W1_e], [b_e1], block=1600, pack=[True])

    Td = jnp.concatenate([Pdf, Pds, Pdt], axis=0)
    idx_x = jnp.stack([src, dst + N], axis=1).reshape(-1).astype(jnp.int32)
    idx_d = jnp.stack([fE, sE + N, tE + 2 * N], axis=1).reshape(-1).astype(jnp.int32)

    Tx1 = jnp.concatenate([Pxs, Pxd], axis=0)
    h1 = _edge_sum(Tx1, Td, idx_x, idx_d, Pe1)

    Wf = (W_e2 @ W1_e).astype(jnp.float32)
    cf = (b_e2 @ W1_e + b_e1).astype(jnp.float32)
    e1, Pe2 = _mm(h1, [W_e2, Wf], [b_e2, cf], block=1600,
                  pack=[False, True])

    agg = jax.ops.segment_sum(h1, agg_idx, num_segments=N)
    agg2 = jax.ops.segment_sum(e1, dstN, num_segments=N)
    G = Qt[tN] + Qf[fN] + Qs[sN]

    Px2s, Px2d = _node_update(x, agg, agg2, G, Wn_x, Wn_a, Wn_a2, b_n,
                              W1_xs, W1_xd)

    Tx2 = jnp.concatenate([Px2s, Px2d], axis=0)
    h2 = _edge_sum(Tx2, Td, idx_x, idx_d, Pe2)
    (e2,) = _mm(h2, [W_e2], [b_e2], block=1600)
    return e2


# speedup vs baseline: 1.1634x; 1.0069x over previous
import functools
import numpy as np
import jax
import jax.numpy as jnp
from jax import lax
from jax.experimental import pallas as pl
from jax.experimental.pallas import tpu as pltpu
from jax.experimental.pallas import tpu_sc as plsc

N, E, D, DE, H = 10000, 160000, 128, 16, 256

_SC_NC, _SC_NS = 2, 16
_NW = _SC_NC * _SC_NS
_EPW = E // _NW
_C = 40
_NCHUNK = _EPW // _C


_PREC = lax.Precision.DEFAULT


def _pack_bf16(acc):
    hw = acc.shape[1] // 2
    lo = lax.bitcast_convert_type(
        acc[:, :hw].astype(jnp.bfloat16), jnp.uint16).astype(jnp.uint32)
    hi = lax.bitcast_convert_type(
        acc[:, hw:].astype(jnp.bfloat16), jnp.uint16).astype(jnp.uint32)
    return lax.bitcast_convert_type(lo | (hi << 16), jnp.int32)


def _mm_body(relu, pack, n_out, a_ref, *refs):
    a = a_ref[...]
    w_refs = refs[:n_out]
    b_refs = refs[n_out:2 * n_out]
    o_refs = refs[2 * n_out:]
    for w_ref, b_ref, o_ref, pk in zip(w_refs, b_refs, o_refs, pack):
        acc = jnp.dot(a, w_ref[...], preferred_element_type=jnp.float32,
                      precision=_PREC) + b_ref[...]
        if relu:
            acc = jnp.maximum(acc, 0.0)
        if pk:
            o_ref[...] = _pack_bf16(acc)
        elif len(o_ref.shape) == 3:
            hw = o_ref.shape[-1]
            o_ref[0] = acc[:, :hw]
            o_ref[1] = acc[:, hw:]
        else:
            o_ref[...] = acc.astype(o_ref.dtype)


def _mm(a, ws, bs, block, relu=False, pack=None, split=None):
    M, K = a.shape
    n_out = len(ws)
    if pack is None:
        pack = [False] * n_out
    if split is None:
        split = [False] * n_out
    grid = (M // block,)
    in_specs = [pl.BlockSpec((block, K), lambda i: (i, 0))]
    in_specs += [pl.BlockSpec(w.shape, lambda i: (0, 0)) for w in ws]
    in_specs += [pl.BlockSpec(b.shape, lambda i: (0,)) for b in bs]
    out_specs = []
    out_shape = []
    for w, pk, sp in zip(ws, pack, split):
        ki = w.shape[1]
        if sp:
            out_specs.append(pl.BlockSpec((2, block, ki // 2),
                                          lambda i: (0, i, 0)))
            out_shape.append(jax.ShapeDtypeStruct((2, M, ki // 2), jnp.float32))
        else:
            kk = ki // 2 if pk else ki
            out_specs.append(pl.BlockSpec((block, kk), lambda i: (i, 0)))
            out_shape.append(jax.ShapeDtypeStruct(
                (M, kk), jnp.int32 if pk else jnp.float32))
    return pl.pallas_call(
        functools.partial(_mm_body, relu, pack, n_out),
        grid=grid,
        in_specs=in_specs,
        out_specs=out_specs,
        out_shape=out_shape,
    )(a, *ws, *bs)


def _mm_packed_body(a_ref, w_ref, b_ref, o_ref):
    ai = a_ref[...]
    lo = lax.bitcast_convert_type(ai << 16, jnp.float32)
    hi = lax.bitcast_convert_type(ai & jnp.int32(-65536), jnp.float32)
    dot = functools.partial(jnp.dot, preferred_element_type=jnp.float32,
                            precision=_PREC)
    o_ref[...] = dot(lo, w_ref[:D]) + dot(hi, w_ref[D:]) + b_ref[...]


def _mm_packed(a, w, b, block):
    M = a.shape[0]
    return pl.pallas_call(
        _mm_packed_body,
        grid=(M // block,),
        in_specs=[pl.BlockSpec((block, D), lambda i: (i, 0)),
                  pl.BlockSpec(w.shape, lambda i: (0, 0)),
                  pl.BlockSpec(b.shape, lambda i: (0,))],
        out_specs=pl.BlockSpec((block, w.shape[1]), lambda i: (i, 0)),
        out_shape=jax.ShapeDtypeStruct((M, w.shape[1]), jnp.float32),
    )(a, w, b)


def _node_update_body(x_ref, agg_ref, agg2_ref, g_ref,
                      wx_ref, wa_ref, wa2_ref, bn_ref, wxs_ref, wxd_ref,
                      oxs_ref, oxd_ref):
    dot = functools.partial(jnp.dot, preferred_element_type=jnp.float32,
                            precision=_PREC)
    v = (dot(x_ref[...], wx_ref[...]) + dot(agg_ref[...], wa_ref[...])
         + dot(agg2_ref[...], wa2_ref[...]) + g_ref[...] + bn_ref[...])
    v = jnp.maximum(v, 0.0)
    oxs_ref[...] = _pack_bf16(dot(v, wxs_ref[...]))
    oxd_ref[...] = _pack_bf16(dot(v, wxd_ref[...]))


def _node_update(x, agg, agg2, g, wx, wa, wa2, bn, wxs, wxd, block=1000):
    grid = (N // block,)
    row = lambda k: pl.BlockSpec((block, k), lambda i: (i, 0))
    full = lambda w: pl.BlockSpec(w.shape, lambda i: (0,) * w.ndim)
    return pl.pallas_call(
        _node_update_body,
        grid=grid,
        in_specs=[row(D), row(H), row(DE), row(D),
                  full(wx), full(wa), full(wa2), full(bn), full(wxs), full(wxd)],
        out_specs=[row(H // 2), row(H // 2)],
        out_shape=[jax.ShapeDtypeStruct((N, H // 2), jnp.int32)] * 2,
    )(x, agg, agg2, g, wx, wa, wa2, bn, wxs, wxd)


def _edge_sum_body(packed_out, tx_hbm, td_hbm, ix_hbm, id_hbm, pe_hbm, out_hbm,
                   ixa, ida, gxs, gds, pebs, ob, sems):
    cid = lax.axis_index("c")
    sid = lax.axis_index("s")
    wid = sid * _SC_NC + cid
    ebase = wid * _EPW

    pltpu.sync_copy(ix_hbm.at[pl.ds(ebase * 2, _EPW * 2)], ixa)
    pltpu.sync_copy(id_hbm.at[pl.ds(ebase * 3, _EPW * 3)], ida)

    def fire(k, b):
        o = k * _C
        pltpu.async_copy(tx_hbm.at[ixa.at[pl.ds(o * 2, 2 * _C)]], gxs[b], sems[b])
        pltpu.async_copy(td_hbm.at[ida.at[pl.ds(o * 3, 3 * _C)]], gds[b], sems[b])
        pltpu.async_copy(pe_hbm.at[pl.ds(ebase + o, _C)], pebs[b], sems[b])

    def wait(b):
        pltpu.make_async_copy(tx_hbm.at[pl.ds(0, 2 * _C)], gxs[b], sems[b]).wait()
        pltpu.make_async_copy(td_hbm.at[pl.ds(0, 3 * _C)], gds[b], sems[b]).wait()
        pltpu.make_async_copy(pe_hbm.at[pl.ds(0, _C)], pebs[b], sems[b]).wait()

    def split(w):
        a = plsc.bitcast(w << 16, jnp.float32)
        b = plsc.bitcast(w & jnp.int32(-65536), jnp.float32)
        return a, b

    def compute(k, b):
        gx, gd, peb = gxs[b], gds[b], pebs[b]

        def row(e, c2):
            for j in range(8):
                s = pl.ds(j * 16, 16)
                pa, pb = split(peb[e, s])
                for r in (gx.at[2 * e], gx.at[2 * e + 1], gd.at[3 * e],
                          gd.at[3 * e + 1], gd.at[3 * e + 2]):
                    ua, ub = split(r[s])
                    pa = pa + ua
                    pb = pb + ub
                pa = jnp.maximum(pa, 0.0)
                pb = jnp.maximum(pb, 0.0)
                if packed_out:
                    ua = plsc.bitcast(pa, jnp.int32) + jnp.int32(0x8000)
                    ub = plsc.bitcast(pb, jnp.int32) + jnp.int32(0x8000)
                    w = ((ua >> 16) & jnp.int32(0xFFFF)) | (ub & jnp.int32(-65536))
                    ob[e, s] = w
                else:
                    ob[e, pl.ds(j * 16, 16)] = pa
                    ob[e, pl.ds(128 + j * 16, 16)] = pb
            return c2

        lax.fori_loop(0, _C, row, 0)
        pltpu.sync_copy(ob, out_hbm.at[pl.ds(ebase + k * _C, _C)])

    fire(0, 0)

    def pair(k2, carry):
        k = 2 * k2
        fire(k + 1, 1)
        wait(0)
        compute(k, 0)
        fire(k + 2, 0)
        wait(1)
        compute(k + 1, 1)
        return carry

    lax.fori_loop(0, (_NCHUNK - 1) // 2, pair, 0)
    wait(0)
    compute(_NCHUNK - 1, 0)


def _edge_sum(tx, td, idx_x, idx_d, pe, packed_out=False):
    mesh = plsc.VectorSubcoreMesh(core_axis_name="c", subcore_axis_name="s",
                                  num_cores=_SC_NC, num_subcores=_SC_NS)
    if packed_out:
        out_type = jax.ShapeDtypeStruct((E, H // 2), jnp.int32)
        ob_type = pltpu.VMEM((_C, H // 2), jnp.int32)
    else:
        out_type = jax.ShapeDtypeStruct((E, H), jnp.float32)
        ob_type = pltpu.VMEM((_C, H), jnp.float32)
    f = pl.kernel(
        functools.partial(_edge_sum_body, packed_out),
        out_type=out_type,
        mesh=mesh,
        compiler_params=pltpu.CompilerParams(needs_layout_passes=False),
        scratch_types=[
            pltpu.VMEM((2 * _EPW,), jnp.int32),
            pltpu.VMEM((3 * _EPW,), jnp.int32),
            [pltpu.VMEM((2 * _C, 128), jnp.int32)] * 2,
            [pltpu.VMEM((3 * _C, 128), jnp.int32)] * 2,
            [pltpu.VMEM((_C, 128), jnp.int32)] * 2,
            ob_type,
            [pltpu.SemaphoreType.DMA] * 2,
        ],
    )
    return f(tx, td, idx_x, idx_d, pe)


_C2T = 80
_C2 = 40


def kernel(x, edge_index, edge_index_NODES, edge_attr, det_features,
           temp_indices_NODES, temp_indices_EDGES,
           first_det_indices_NODES, first_det_indices_EDGES,
           second_det_indices_NODES, second_det_indices_EDGES,
           indices_for_aggregating_nodes_updates,
           W_e1, b_e1, W_e2, b_e2, W_n, b_n):
    src, dst = edge_index[0], edge_index[1]
    dstN = edge_index_NODES[1]
    fE, sE, tE = (first_det_indices_EDGES, second_det_indices_EDGES,
                  temp_indices_EDGES)
    fN, sN, tN = (first_det_indices_NODES, second_det_indices_NODES,
                  temp_indices_NODES)
    agg_idx = indices_for_aggregating_nodes_updates

    W1_xs, W1_xd = W_e1[0:D], W_e1[D:2*D]
    W1_e = W_e1[2*D:2*D+DE]
    W1_f, W1_s, W1_t = (W_e1[2*D+DE:3*D+DE], W_e1[3*D+DE:4*D+DE],
                        W_e1[4*D+DE:5*D+DE])
    Wn_x, Wn_a, Wn_a2 = W_n[0:D], W_n[D:D+H], W_n[D+H:D+H+DE]
    Wn_t, Wn_f, Wn_s = (W_n[D+H+DE:2*D+H+DE], W_n[2*D+H+DE:3*D+H+DE],
                        W_n[3*D+H+DE:4*D+H+DE])
    zH = jnp.zeros((H,), jnp.float32)
    zD = jnp.zeros((D,), jnp.float32)
    zE = jnp.zeros((DE,), jnp.float32)

    Pdf, Pds, Pdt, Qt, Qf, Qs = _mm(
        det_features, [W1_f, W1_s, W1_t, Wn_t, Wn_f, Wn_s],
        [zH, zH, zH, zD, zD, zD], block=1000,
        pack=[True, True, True, False, False, False])
    Pxs, Pxd = _mm(x, [W1_xs, W1_xd], [zH, zH], block=1000, pack=[True, True])
    (Pe1,) = _mm(edge_attr, [W1_e], [b_e1], block=1600, pack=[True])

    Td = jnp.concatenate([Pdf, Pds, Pdt], axis=0)
    idx_x = jnp.stack([src, dst + N], axis=1).reshape(-1).astype(jnp.int32)
    idx_d = jnp.stack([fE, sE + N, tE + 2 * N], axis=1).reshape(-1).astype(jnp.int32)

    Tx1 = jnp.concatenate([Pxs, Pxd], axis=0)
    h1 = _edge_sum(Tx1, Td, idx_x, idx_d, Pe1)

    Wf = (W_e2 @ W1_e).astype(jnp.float32)
    cf = (b_e2 @ W1_e + b_e1).astype(jnp.float32)
    e1, Pe2 = _mm(h1, [W_e2, Wf], [b_e2, cf], block=1600,
                  pack=[False, True])

    agg = jax.ops.segment_sum(h1, agg_idx, num_segments=N)
    agg2 = jax.ops.segment_sum(e1, dstN, num_segments=N)
    G = Qt[tN] + Qf[fN] + Qs[sN]

    Px2s, Px2d = _node_update(x, agg, agg2, G, Wn_x, Wn_a, Wn_a2, b_n,
                              W1_xs, W1_xd)

    Tx2 = jnp.concatenate([Px2s, Px2d], axis=0)
    h2 = _edge_sum(Tx2, Td, idx_x, idx_d, Pe2, packed_out=True)
    e2 = _mm_packed(h2, W_e2, b_e2, block=1600)
    return e2
